# baseline (speedup 1.0000x reference)
import functools

import jax
import jax.numpy as jnp
from jax import lax
from jax.experimental import pallas as pl
from jax.experimental.pallas import tpu as pltpu
from jax.experimental.pallas import tpu_sc as plsc

_K = 16
_D = 128
_N = 2048
_B = 4
_BLK = 512
_LN = 128
_S = 5
_LANES = 16
_CP = 8


def _prep_body(x_ref, w_ref, bb_ref, y_ref, z_ref):
    xb = x_ref[0]
    a = w_ref[:, :_D]
    dm = w_ref[:, _D:] - a
    y_ref[0] = lax.dot_general(
        xb, a, (((0,), (1,)), ((), ())),
        preferred_element_type=jnp.float32,
        precision=lax.Precision.HIGHEST)
    z_ref[0] = lax.dot_general(
        dm, xb, (((1,), (0,)), ((), ())),
        preferred_element_type=jnp.float32,
        precision=lax.Precision.HIGHEST) + bb_ref[...]


def _knn_body(xfull_ref, xblk_ref, idx_ref, *, base):
    xb = xfull_ref[0]
    xblk = xblk_ref[0]
    inner = lax.dot_general(
        xblk, xb, (((0,), (0,)), ((), ())),
        preferred_element_type=jnp.float32,
        precision=lax.Precision.DEFAULT)

    xx = jnp.sum(xb * xb, axis=0, keepdims=True)
    score = 2.0 * inner - xx
    neg = jnp.float32(float("-inf"))
    big = jnp.float32(1.0e9)
    npages = _N // _LN
    lanesf = lax.broadcasted_iota(
        jnp.int32, (_BLK, _LN), 1).astype(jnp.float32)

    f = [jnp.full((_BLK, _LN), neg, jnp.float32) for _ in range(_S)]
    p = [jnp.zeros((_BLK, _LN), jnp.float32) for _ in range(_S)]
    for q in range(npages):
        v = score[:, q * _LN:(q + 1) * _LN]
        qf = jnp.float32(q)
        c = [v > f[k] for k in range(_S)]
        for k in range(_S - 1, 0, -1):
            f[k] = jnp.where(c[k], jnp.where(c[k - 1], f[k - 1], v), f[k])
            p[k] = jnp.where(c[k], jnp.where(c[k - 1], p[k - 1], qf), p[k])
        f[0] = jnp.where(c[0], v, f[0])
        p[0] = jnp.where(c[0], qf, p[0])

    outs = []
    for _ in range(_K):
        m = jnp.max(f[0], axis=1, keepdims=True)
        lane_cand = jnp.where(f[0] == m, lanesf, big)
        alf = jnp.min(lane_cand, axis=1, keepdims=True)
        pop = lanesf == alf
        pg = jnp.min(jnp.where(pop, p[0], big), axis=1, keepdims=True)
        outs.append(pg * jnp.float32(_LN) + alf)
        for k in range(_S - 1):
            f[k] = jnp.where(pop, f[k + 1], f[k])
            p[k] = jnp.where(pop, p[k + 1], p[k])
        f[_S - 1] = jnp.where(pop, neg, f[_S - 1])
    idxf = jnp.concatenate(outs, axis=1)
    idx_ref[...] = idxf.astype(jnp.int32) + base

    @pl.when(jnp.min(f[0]) < jnp.float32(-1.0e37))
    def _fallback():
        colsf = lax.broadcasted_iota(
            jnp.int32, (_BLK, _N), 1).astype(jnp.float32)
        sc = score
        outs2 = []
        for _ in range(_K):
            m2 = jnp.max(sc, axis=1, keepdims=True)
            cand = jnp.where(sc == m2, colsf, big)
            amf = jnp.min(cand, axis=1, keepdims=True)
            outs2.append(amf)
            sc = jnp.where(cand == amf, neg, sc)
        idxf2 = jnp.concatenate(outs2, axis=1)
        idx_ref[...] = idxf2.astype(jnp.int32) + base


def _finish_body(m_ref, z_ref, o_ref):
    mb = m_ref[...]
    i0 = lax.broadcasted_iota(jnp.int32, (_D, _D), 0)
    i1 = lax.broadcasted_iota(jnp.int32, (_D, _D), 1)
    eye = (i0 == i1).astype(jnp.float32)
    t = lax.dot_general(
        mb, eye, (((0,), (0,)), ((), ())),
        preferred_element_type=jnp.float32,
        precision=lax.Precision.HIGHEST)
    o_ref[...] = jnp.maximum(t + z_ref[0], 0.0)


@functools.lru_cache(maxsize=None)
def _make_gathermax(p):
    info = plsc.get_sparse_core_info()
    nc = info.num_cores
    nw = nc * info.num_subcores
    ppw = p // nw
    nch = ppw // _CP
    mesh = plsc.VectorSubcoreMesh(core_axis_name="c", subcore_axis_name="s")

    nbuf = 4

    @functools.partial(
        pl.kernel,
        mesh=mesh,
        out_type=jax.ShapeDtypeStruct((p, _D), jnp.float32),
        scratch_types=[
            pltpu.VMEM((nbuf, _CP * _K), jnp.int32),
            pltpu.VMEM((nbuf, _CP * _K, _D), jnp.float32),
            pltpu.VMEM((ppw, _D), jnp.float32),
            pltpu.SemaphoreType.DMA,
            pltpu.SemaphoreType.DMA,
            pltpu.SemaphoreType.DMA,
            pltpu.SemaphoreType.DMA,
        ],
    )
    def gathermax(y_hbm, gidx_hbm, m_hbm, idx_v, rows_v, out_v, *sems):
        wid = lax.axis_index("s") * nc + lax.axis_index("c")
        base = wid * ppw

        def issue(g, slot):
            off = (base + g * _CP) * _K
            pltpu.sync_copy(gidx_hbm.at[pl.ds(off, _CP * _K)], idx_v.at[slot])
            pltpu.make_async_copy(
                y_hbm.at[idx_v.at[slot]], rows_v.at[slot], sems[slot]).start()

        for slot in range(nbuf):
            issue(slot, slot)

        def outer(i, carry):
            for slot in range(nbuf):
                g = i * nbuf + slot
                pltpu.make_async_copy(
                    y_hbm.at[idx_v.at[slot]], rows_v.at[slot],
                    sems[slot]).wait()

                def ptbody(pt, c2, slot=slot, g=g):
                    for c in range(_D // _LANES):
                        sl = pl.ds(c * _LANES, _LANES)
                        vals = [rows_v[slot, pt * _K + r, sl]
                                for r in range(_K)]
                        while len(vals) > 1:
                            vals = [jnp.maximum(vals[i2], vals[i2 + 1])
                                    if i2 + 1 < len(vals) else vals[i2]
                                    for i2 in range(0, len(vals), 2)]
                        out_v[g * _CP + pt, sl] = vals[0]
                    return c2

                lax.fori_loop(0, _CP, ptbody, 0)
                gn = g + nbuf
                @pl.when(gn < nch)
                def _():
                    issue(gn, slot)
            return carry

        lax.fori_loop(0, nch // nbuf, outer, 0)
        pltpu.sync_copy(out_v, m_hbm.at[pl.ds(base, ppw)])

    return gathermax


def kernel(x, W, b):
    bb = b.reshape(_D, 1)
    y, z = pl.pallas_call(
        _prep_body,
        grid=(_B,),
        in_specs=[
            pl.BlockSpec((1, _D, _N), lambda i: (i, 0, 0)),
            pl.BlockSpec((_D, 2 * _D), lambda i: (0, 0)),
            pl.BlockSpec((_D, 1), lambda i: (0, 0)),
        ],
        out_specs=[
            pl.BlockSpec((1, _N, _D), lambda i: (i, 0, 0)),
            pl.BlockSpec((1, _D, _N), lambda i: (i, 0, 0)),
        ],
        out_shape=[
            jax.ShapeDtypeStruct((_B, _N, _D), jnp.float32),
            jax.ShapeDtypeStruct((_B, _D, _N), jnp.float32),
        ],
    )(x, W, bb)

    y_flat = y.reshape(_B * _N, _D)
    gm = _make_gathermax(_N)
    outs = []
    for bq in range(_B):
        idx_b = pl.pallas_call(
            functools.partial(_knn_body, base=bq * _N),
            grid=(_N // _BLK,),
            in_specs=[
                pl.BlockSpec((1, _D, _N), lambda i, bq=bq: (bq, 0, 0)),
                pl.BlockSpec((1, _D, _BLK), lambda i, bq=bq: (bq, 0, i)),
            ],
            out_specs=pl.BlockSpec((_BLK, _K), lambda i: (i, 0)),
            out_shape=jax.ShapeDtypeStruct((_N, _K), jnp.int32),
        )(x, x)
        m_b = gm(y_flat, idx_b.reshape(_N * _K))
        out_b = pl.pallas_call(
            _finish_body,
            grid=(_N // _D,),
            in_specs=[
                pl.BlockSpec((_D, _D), lambda i: (i, 0)),
                pl.BlockSpec((1, _D, _D), lambda i, bq=bq: (bq, 0, i)),
            ],
            out_specs=pl.BlockSpec((_D, _D), lambda i: (0, i)),
            out_shape=jax.ShapeDtypeStruct((_D, _N), jnp.float32),
        )(m_b, z)
        outs.append(out_b)
    return jnp.stack(outs)


# speedup vs baseline: 1.1086x; 1.1086x over previous
import functools

import jax
import jax.numpy as jnp
from jax import lax
from jax.experimental import pallas as pl
from jax.experimental.pallas import tpu as pltpu
from jax.experimental.pallas import tpu_sc as plsc

_K = 16
_D = 128
_N = 2048
_B = 4
_BLK = 256
_LN = 128
_S = 5
_LANES = 16
_CP = 8


def _prep_body(x_ref, w_ref, bb_ref, y_ref, z_ref):
    xb = x_ref[0]
    a = w_ref[:, :_D]
    dm = w_ref[:, _D:] - a
    y_ref[0] = lax.dot_general(
        xb, a, (((0,), (1,)), ((), ())),
        preferred_element_type=jnp.float32,
        precision=lax.Precision.HIGHEST)
    z_ref[0] = lax.dot_general(
        dm, xb, (((1,), (0,)), ((), ())),
        preferred_element_type=jnp.float32,
        precision=lax.Precision.HIGHEST) + bb_ref[...]


def _knn_body(xfull_ref, xblk_ref, idx_ref, *, base):
    xb = xfull_ref[0]
    xblk = xblk_ref[0]
    inner = lax.dot_general(
        xblk, xb, (((0,), (0,)), ((), ())),
        preferred_element_type=jnp.float32,
        precision=lax.Precision.DEFAULT)

    xx = jnp.sum(xb * xb, axis=0, keepdims=True)
    score = 2.0 * inner - xx
    neg = jnp.float32(float("-inf"))
    big = jnp.float32(1.0e9)
    npages = _N // _LN
    lanesf = lax.broadcasted_iota(
        jnp.int32, (_BLK, _LN), 1).astype(jnp.float32)

    f = [jnp.full((_BLK, _LN), neg, jnp.float32) for _ in range(_S)]
    p = [jnp.zeros((_BLK, _LN), jnp.float32) for _ in range(_S)]
    for q in range(npages):
        v = score[:, q * _LN:(q + 1) * _LN]
        qf = jnp.float32(q)
        c = [v > f[k] for k in range(_S)]
        for k in range(_S - 1, 0, -1):
            f[k] = jnp.where(c[k], jnp.where(c[k - 1], f[k - 1], v), f[k])
            p[k] = jnp.where(c[k], jnp.where(c[k - 1], p[k - 1], qf), p[k])
        f[0] = jnp.where(c[0], v, f[0])
        p[0] = jnp.where(c[0], qf, p[0])

    outs = []
    for _ in range(_K):
        m = jnp.max(f[0], axis=1, keepdims=True)
        lane_cand = jnp.where(f[0] == m, lanesf, big)
        alf = jnp.min(lane_cand, axis=1, keepdims=True)
        pop = lanesf == alf
        pg = jnp.min(jnp.where(pop, p[0], big), axis=1, keepdims=True)
        outs.append(pg * jnp.float32(_LN) + alf)
        for k in range(_S - 1):
            f[k] = jnp.where(pop, f[k + 1], f[k])
            p[k] = jnp.where(pop, p[k + 1], p[k])
        f[_S - 1] = jnp.where(pop, neg, f[_S - 1])
    idxf = jnp.concatenate(outs, axis=1)
    idx_ref[...] = idxf.astype(jnp.int32) + base

    @pl.when(jnp.min(f[0]) < jnp.float32(-1.0e37))
    def _fallback():
        colsf = lax.broadcasted_iota(
            jnp.int32, (_BLK, _N), 1).astype(jnp.float32)
        sc = score
        outs2 = []
        for _ in range(_K):
            m2 = jnp.max(sc, axis=1, keepdims=True)
            cand = jnp.where(sc == m2, colsf, big)
            amf = jnp.min(cand, axis=1, keepdims=True)
            outs2.append(amf)
            sc = jnp.where(cand == amf, neg, sc)
        idxf2 = jnp.concatenate(outs2, axis=1)
        idx_ref[...] = idxf2.astype(jnp.int32) + base


def _finish_body(m_ref, z_ref, o_ref):
    mb = m_ref[...]
    i0 = lax.broadcasted_iota(jnp.int32, (_D, _D), 0)
    i1 = lax.broadcasted_iota(jnp.int32, (_D, _D), 1)
    eye = (i0 == i1).astype(jnp.float32)
    t = lax.dot_general(
        mb, eye, (((0,), (0,)), ((), ())),
        preferred_element_type=jnp.float32,
        precision=lax.Precision.HIGHEST)
    o_ref[...] = jnp.maximum(t + z_ref[0], 0.0)


@functools.lru_cache(maxsize=None)
def _make_gathermax(p):
    info = plsc.get_sparse_core_info()
    nc = info.num_cores
    nw = nc * info.num_subcores
    ppw = p // nw
    nch = ppw // _CP
    mesh = plsc.VectorSubcoreMesh(core_axis_name="c", subcore_axis_name="s")

    nbuf = 4

    @functools.partial(
        pl.kernel,
        mesh=mesh,
        out_type=jax.ShapeDtypeStruct((p, _D), jnp.float32),
        scratch_types=[
            pltpu.VMEM((nbuf, _CP * _K), jnp.int32),
            pltpu.VMEM((nbuf, _CP * _K, _D), jnp.float32),
            pltpu.VMEM((ppw, _D), jnp.float32),
            pltpu.SemaphoreType.DMA,
            pltpu.SemaphoreType.DMA,
            pltpu.SemaphoreType.DMA,
            pltpu.SemaphoreType.DMA,
        ],
    )
    def gathermax(y_hbm, gidx_hbm, m_hbm, idx_v, rows_v, out_v, *sems):
        wid = lax.axis_index("s") * nc + lax.axis_index("c")
        base = wid * ppw

        def issue(g, slot):
            off = (base + g * _CP) * _K
            pltpu.sync_copy(gidx_hbm.at[pl.ds(off, _CP * _K)], idx_v.at[slot])
            pltpu.make_async_copy(
                y_hbm.at[idx_v.at[slot]], rows_v.at[slot], sems[slot]).start()

        for slot in range(nbuf):
            issue(slot, slot)

        def outer(i, carry):
            for slot in range(nbuf):
                g = i * nbuf + slot
                pltpu.make_async_copy(
                    y_hbm.at[idx_v.at[slot]], rows_v.at[slot],
                    sems[slot]).wait()

                def ptbody(pt, c2, slot=slot, g=g):
                    for c in range(_D // _LANES):
                        sl = pl.ds(c * _LANES, _LANES)
                        vals = [rows_v[slot, pt * _K + r, sl]
                                for r in range(_K)]
                        while len(vals) > 1:
                            vals = [jnp.maximum(vals[i2], vals[i2 + 1])
                                    if i2 + 1 < len(vals) else vals[i2]
                                    for i2 in range(0, len(vals), 2)]
                        out_v[g * _CP + pt, sl] = vals[0]
                    return c2

                lax.fori_loop(0, _CP, ptbody, 0)
                gn = g + nbuf
                @pl.when(gn < nch)
                def _():
                    issue(gn, slot)
            return carry

        lax.fori_loop(0, nch // nbuf, outer, 0)
        pltpu.sync_copy(out_v, m_hbm.at[pl.ds(base, ppw)])

    return gathermax


def kernel(x, W, b):
    bb = b.reshape(_D, 1)
    y, z = pl.pallas_call(
        _prep_body,
        grid=(_B,),
        in_specs=[
            pl.BlockSpec((1, _D, _N), lambda i: (i, 0, 0)),
            pl.BlockSpec((_D, 2 * _D), lambda i: (0, 0)),
            pl.BlockSpec((_D, 1), lambda i: (0, 0)),
        ],
        out_specs=[
            pl.BlockSpec((1, _N, _D), lambda i: (i, 0, 0)),
            pl.BlockSpec((1, _D, _N), lambda i: (i, 0, 0)),
        ],
        out_shape=[
            jax.ShapeDtypeStruct((_B, _N, _D), jnp.float32),
            jax.ShapeDtypeStruct((_B, _D, _N), jnp.float32),
        ],
    )(x, W, bb)

    y_flat = y.reshape(_B * _N, _D)
    gm = _make_gathermax(_N)
    outs = []
    for bq in range(_B):
        idx_b = pl.pallas_call(
            functools.partial(_knn_body, base=bq * _N),
            grid=(_N // _BLK,),
            in_specs=[
                pl.BlockSpec((1, _D, _N), lambda i, bq=bq: (bq, 0, 0)),
                pl.BlockSpec((1, _D, _BLK), lambda i, bq=bq: (bq, 0, i)),
            ],
            out_specs=pl.BlockSpec((_BLK, _K), lambda i: (i, 0)),
            out_shape=jax.ShapeDtypeStruct((_N, _K), jnp.int32),
        )(x, x)
        m_b = gm(y_flat, idx_b.reshape(_N * _K))
        out_b = pl.pallas_call(
            _finish_body,
            grid=(_N // _D,),
            in_specs=[
                pl.BlockSpec((_D, _D), lambda i: (i, 0)),
                pl.BlockSpec((1, _D, _D), lambda i, bq=bq: (bq, 0, i)),
            ],
            out_specs=pl.BlockSpec((_D, _D), lambda i: (0, i)),
            out_shape=jax.ShapeDtypeStruct((_D, _N), jnp.float32),
        )(m_b, z)
        outs.append(out_b)
    return jnp.stack(outs)
